# Initial kernel scaffold; baseline (speedup 1.0000x reference)
import jax
import jax.numpy as jnp
from jax.experimental import pallas as pl


def kernel(pair_0, pair_1, target_table, context_table):
    raise NotImplementedError("write your pallas kernel here")


# speedup vs baseline: 5.5143x; 5.5143x over previous
import functools

import jax
import jax.numpy as jnp
from jax import lax
from jax.experimental import pallas as pl
from jax.experimental.pallas import tpu as pltpu
from jax.experimental.pallas import tpu_sc as plsc

V = 1000
D = 64
B = 16384
NCTX = 5

NUM_SC = 2
NUM_TEC = 16
NW = NUM_SC * NUM_TEC
PAIRS_W = B // NW
OUT_W = PAIRS_W * NCTX
ROW = 128
NROWS = OUT_W // ROW


def _matmul_body(c_ref, t_ref, m_ref):
    m_ref[...] = lax.dot_general(
        c_ref[...],
        t_ref[...],
        dimension_numbers=(((1,), (1,)), ((), ())),
        preferred_element_type=jnp.float32,
    )


def _pairwise_dots(context_table, target_table):
    return pl.pallas_call(
        _matmul_body,
        out_shape=jax.ShapeDtypeStruct((V, V), jnp.float32),
    )(context_table, target_table)


_sc_mesh = plsc.VectorSubcoreMesh(core_axis_name="c", subcore_axis_name="s")


@functools.partial(
    pl.kernel,
    out_type=jax.ShapeDtypeStruct((B * NCTX,), jnp.float32),
    mesh=_sc_mesh,
    scratch_types=[
        pltpu.VMEM((OUT_W,), jnp.int32),
        pltpu.VMEM((OUT_W,), jnp.int32),
        pltpu.VMEM((NROWS, ROW), jnp.int32),
        pltpu.VMEM((OUT_W,), jnp.float32),
        pltpu.SemaphoreType.DMA,
    ],
)
def _sc_gather(p0_hbm, p1_hbm, m_hbm, out_hbm, p0_v, p1_v, idx_v, vals_v, sem):
    wid = lax.axis_index("s") * NUM_SC + lax.axis_index("c")
    base_k = wid * OUT_W
    pltpu.sync_copy(p0_hbm.at[pl.ds(base_k, OUT_W)], p0_v)
    pltpu.sync_copy(p1_hbm.at[pl.ds(base_k, OUT_W)], p1_v)

    def fire(j, carry):
        for c in range(ROW // 16):
            k0 = j * ROW + c * 16
            tvals = p0_v[pl.ds(k0, 16)]
            cvals = p1_v[pl.ds(k0, 16)]
            idx_v[j, pl.ds(c * 16, 16)] = cvals * V + tvals
        pltpu.async_copy(m_hbm.at[idx_v.at[j]], vals_v.at[pl.ds(j * ROW, ROW)], sem)
        return carry

    lax.fori_loop(0, NROWS, fire, 0)

    def drain(j, carry):
        pltpu.make_async_copy(
            m_hbm.at[idx_v.at[j]], vals_v.at[pl.ds(j * ROW, ROW)], sem
        ).wait()
        return carry

    lax.fori_loop(0, NROWS, drain, 0)
    pltpu.sync_copy(vals_v, out_hbm.at[pl.ds(base_k, OUT_W)])


def kernel(pair_0, pair_1, target_table, context_table):
    m = _pairwise_dots(context_table, target_table)
    p0_rep = jnp.broadcast_to(pair_0.reshape(B, 1), (B, NCTX)).reshape(-1)
    out = _sc_gather(p0_rep, pair_1.reshape(-1), m.reshape(-1))
    return out.reshape(B, NCTX)


# speedup vs baseline: 12.4329x; 2.2547x over previous
import functools

import jax
import jax.numpy as jnp
from jax import lax
from jax.experimental import pallas as pl
from jax.experimental.pallas import tpu as pltpu
from jax.experimental.pallas import tpu_sc as plsc

V = 1000
D = 64
B = 16384
NCTX = 5

GROW = 128
NG = 8

NUM_SC = 2
NUM_TEC = 16
NW = NUM_SC * NUM_TEC
PAIRS_W = B // NW
OUT_W = PAIRS_W * NCTX
ROW = 128
JROWS = PAIRS_W // ROW


def _matmul_body(c_ref, t_ref, m_ref):
    m_ref[...] = lax.dot_general(
        c_ref[...],
        t_ref[...],
        dimension_numbers=(((1,), (1,)), ((), ())),
        preferred_element_type=jnp.float32,
    )


def _pairwise_dots(context_table, target_table):
    return pl.pallas_call(
        _matmul_body,
        grid=(NG,),
        in_specs=[
            pl.BlockSpec((V, D), lambda g: (0, 0)),
            pl.BlockSpec((GROW, D), lambda g: (g, 0)),
        ],
        out_specs=pl.BlockSpec((V, GROW), lambda g: (g, 0)),
        out_shape=jax.ShapeDtypeStruct((NG * V, GROW), jnp.float32),
    )(context_table, target_table)


_sc_mesh = plsc.VectorSubcoreMesh(core_axis_name="c", subcore_axis_name="s")


@functools.partial(
    pl.kernel,
    out_type=jax.ShapeDtypeStruct((NCTX * B,), jnp.float32),
    mesh=_sc_mesh,
    scratch_types=[
        pltpu.VMEM((PAIRS_W,), jnp.int32),
        pltpu.VMEM((OUT_W,), jnp.int32),
        pltpu.VMEM((NCTX * JROWS, ROW), jnp.int32),
        pltpu.VMEM((OUT_W,), jnp.float32),
        pltpu.SemaphoreType.DMA,
    ],
)
def _sc_gather(p0_hbm, p1t_hbm, m_hbm, out_hbm, p0_v, p1_v, idx_v, vals_v, sem):
    wid = lax.axis_index("s") * NUM_SC + lax.axis_index("c")
    base_b = wid * PAIRS_W
    pltpu.sync_copy(p0_hbm.at[pl.ds(base_b, PAIRS_W)], p0_v)
    for n in range(NCTX):
        pltpu.sync_copy(
            p1t_hbm.at[pl.ds(n * B + base_b, PAIRS_W)],
            p1_v.at[pl.ds(n * PAIRS_W, PAIRS_W)],
        )

    for n in range(NCTX):

        def fire(j, carry, n=n):
            for c in range(ROW // 16):
                k0 = j * ROW + c * 16
                it = p0_v[pl.ds(k0, 16)]
                jc = p1_v[pl.ds(n * PAIRS_W + k0, 16)]
                idx_v[n * JROWS + j, pl.ds(c * 16, 16)] = (
                    lax.shift_right_logical(it, 7) * (V * GROW)
                    + jc * GROW
                    + lax.bitwise_and(it, GROW - 1)
                )
            pltpu.async_copy(
                m_hbm.at[idx_v.at[n * JROWS + j]],
                vals_v.at[pl.ds(n * PAIRS_W + j * ROW, ROW)],
                sem,
            )
            return carry

        lax.fori_loop(0, JROWS, fire, 0)

    for n in range(NCTX):

        def drain(j, carry, n=n):
            pltpu.make_async_copy(
                m_hbm.at[idx_v.at[n * JROWS + j]],
                vals_v.at[pl.ds(n * PAIRS_W + j * ROW, ROW)],
                sem,
            ).wait()
            return carry

        lax.fori_loop(0, JROWS, drain, 0)

    for n in range(NCTX):
        pltpu.sync_copy(
            vals_v.at[pl.ds(n * PAIRS_W, PAIRS_W)],
            out_hbm.at[pl.ds(n * B + base_b, PAIRS_W)],
        )


def kernel(pair_0, pair_1, target_table, context_table):
    m2 = _pairwise_dots(context_table, target_table)
    p0_flat = pair_0.reshape(-1)
    p1_t = pair_1.transpose(1, 2, 0).reshape(-1)
    out_t = _sc_gather(p0_flat, p1_t, m2.reshape(-1))
    return out_t.reshape(NCTX, B).T


# speedup vs baseline: 15.0642x; 1.2116x over previous
import functools

import jax
import jax.numpy as jnp
from jax import lax
from jax.experimental import pallas as pl
from jax.experimental.pallas import tpu as pltpu
from jax.experimental.pallas import tpu_sc as plsc

V = 1000
D = 64
B = 16384
NCTX = 5

GROW = 128
NG = 8

NUM_SC = 2
NUM_TEC = 16
NW = NUM_SC * NUM_TEC
PAIRS_W = B // NW
OUT_W = PAIRS_W * NCTX
ROW = 128
JROWS = PAIRS_W // ROW


def _matmul_body(ct_ref, tt_ref, m_ref):
    c = ct_ref[...]
    for g in range(NG):
        w = min(GROW, V - g * GROW)
        t_g = tt_ref[:, g * GROW : g * GROW + w]
        m_ref[pl.ds(g * V, V), 0:w] = lax.dot_general(
            c,
            t_g,
            dimension_numbers=(((0,), (0,)), ((), ())),
            preferred_element_type=jnp.float32,
        )


def _pairwise_dots(context_table_t, target_table_t):
    return pl.pallas_call(
        _matmul_body,
        out_shape=jax.ShapeDtypeStruct((NG * V, GROW), jnp.float32),
    )(context_table_t, target_table_t)


_sc_mesh = plsc.VectorSubcoreMesh(core_axis_name="c", subcore_axis_name="s")


@functools.partial(
    pl.kernel,
    out_type=jax.ShapeDtypeStruct((NCTX * B,), jnp.float32),
    mesh=_sc_mesh,
    scratch_types=[
        pltpu.VMEM((PAIRS_W,), jnp.int32),
        pltpu.VMEM((OUT_W,), jnp.int32),
        pltpu.VMEM((NCTX * JROWS, ROW), jnp.int32),
        pltpu.VMEM((OUT_W,), jnp.float32),
        pltpu.SemaphoreType.DMA,
    ],
)
def _sc_gather(p0_hbm, p1t_hbm, m_hbm, out_hbm, p0_v, p1_v, idx_v, vals_v, sem):
    wid = lax.axis_index("s") * NUM_SC + lax.axis_index("c")
    base_b = wid * PAIRS_W
    pltpu.sync_copy(p0_hbm.at[pl.ds(base_b, PAIRS_W)], p0_v)
    for n in range(NCTX):
        pltpu.sync_copy(
            p1t_hbm.at[pl.ds(n * B + base_b, PAIRS_W)],
            p1_v.at[pl.ds(n * PAIRS_W, PAIRS_W)],
        )

    for n in range(NCTX):

        def fire(j, carry, n=n):
            for c in range(ROW // 16):
                k0 = j * ROW + c * 16
                it = p0_v[pl.ds(k0, 16)]
                jc = p1_v[pl.ds(n * PAIRS_W + k0, 16)]
                idx_v[n * JROWS + j, pl.ds(c * 16, 16)] = (
                    lax.shift_right_logical(it, 7) * (V * GROW)
                    + jc * GROW
                    + lax.bitwise_and(it, GROW - 1)
                )
            pltpu.async_copy(
                m_hbm.at[idx_v.at[n * JROWS + j]],
                vals_v.at[pl.ds(n * PAIRS_W + j * ROW, ROW)],
                sem,
            )
            return carry

        lax.fori_loop(0, JROWS, fire, 0)

    for n in range(NCTX):

        def drain(j, carry, n=n):
            pltpu.make_async_copy(
                m_hbm.at[idx_v.at[n * JROWS + j]],
                vals_v.at[pl.ds(n * PAIRS_W + j * ROW, ROW)],
                sem,
            ).wait()
            return carry

        lax.fori_loop(0, JROWS, drain, 0)

    for n in range(NCTX):
        pltpu.sync_copy(
            vals_v.at[pl.ds(n * PAIRS_W, PAIRS_W)],
            out_hbm.at[pl.ds(n * B + base_b, PAIRS_W)],
        )


def kernel(pair_0, pair_1, target_table, context_table):
    m2 = _pairwise_dots(context_table.T, target_table.T)
    p0_flat = pair_0.reshape(-1)
    p1_t = pair_1.transpose(1, 2, 0).reshape(-1)
    out_t = _sc_gather(p0_flat, p1_t, m2.reshape(-1))
    return out_t.reshape(NCTX, B).T


# speedup vs baseline: 16.5148x; 1.0963x over previous
import functools

import jax
import jax.numpy as jnp
from jax import lax
from jax.experimental import pallas as pl
from jax.experimental.pallas import tpu as pltpu
from jax.experimental.pallas import tpu_sc as plsc

V = 1000
D = 64
B = 16384
NCTX = 5

GROW = 128
NG = 8

NUM_SC = 2
NUM_TEC = 16
NW = NUM_SC * NUM_TEC
PAIRS_W = B // NW
OUT_W = PAIRS_W * NCTX
ROW = 128
JROWS = PAIRS_W // ROW


def _matmul_body(ct_ref, tt_ref, m_ref):
    c = ct_ref[...]
    for g in range(NG):
        w = min(GROW, V - g * GROW)
        t_g = tt_ref[:, g * GROW : g * GROW + w]
        m_ref[pl.ds(g * V, V), 0:w] = lax.dot_general(
            c,
            t_g,
            dimension_numbers=(((0,), (0,)), ((), ())),
            preferred_element_type=jnp.float32,
        )


def _pairwise_dots(context_table_t, target_table_t):
    return pl.pallas_call(
        _matmul_body,
        out_shape=jax.ShapeDtypeStruct((NG * V, GROW), jnp.float32),
    )(context_table_t, target_table_t)


_sc_mesh = plsc.VectorSubcoreMesh(core_axis_name="c", subcore_axis_name="s")


@functools.partial(
    pl.kernel,
    out_type=jax.ShapeDtypeStruct((NCTX * B,), jnp.float32),
    mesh=_sc_mesh,
    scratch_types=[
        pltpu.VMEM((PAIRS_W,), jnp.int32),
        pltpu.VMEM((OUT_W,), jnp.int32),
        pltpu.VMEM((NCTX * JROWS, ROW), jnp.int32),
        pltpu.VMEM((OUT_W,), jnp.float32),
        pltpu.SemaphoreType.DMA,
        pltpu.SemaphoreType.DMA,
        pltpu.SemaphoreType.DMA,
    ],
)
def _sc_gather(p0_hbm, p1t_hbm, m_hbm, out_hbm, p0_v, p1_v, idx_v, vals_v,
               sem, sem_in, sem_out):
    wid = lax.axis_index("s") * NUM_SC + lax.axis_index("c")
    base_b = wid * PAIRS_W
    pltpu.async_copy(p0_hbm.at[pl.ds(base_b, PAIRS_W)], p0_v, sem_in)
    for n in range(NCTX):
        pltpu.async_copy(
            p1t_hbm.at[pl.ds(n * B + base_b, PAIRS_W)],
            p1_v.at[pl.ds(n * PAIRS_W, PAIRS_W)],
            sem_in,
        )
    pltpu.make_async_copy(p0_hbm.at[pl.ds(base_b, PAIRS_W)], p0_v, sem_in).wait()
    for n in range(NCTX):
        pltpu.make_async_copy(
            p1t_hbm.at[pl.ds(n * B + base_b, PAIRS_W)],
            p1_v.at[pl.ds(n * PAIRS_W, PAIRS_W)],
            sem_in,
        ).wait()

    for n in range(NCTX):

        def fire(j, carry, n=n):
            for c in range(ROW // 16):
                k0 = j * ROW + c * 16
                it = p0_v[pl.ds(k0, 16)]
                jc = p1_v[pl.ds(n * PAIRS_W + k0, 16)]
                idx_v[n * JROWS + j, pl.ds(c * 16, 16)] = (
                    lax.shift_right_logical(it, 7) * (V * GROW)
                    + jc * GROW
                    + lax.bitwise_and(it, GROW - 1)
                )
            pltpu.async_copy(
                m_hbm.at[idx_v.at[n * JROWS + j]],
                vals_v.at[pl.ds(n * PAIRS_W + j * ROW, ROW)],
                sem,
            )
            return carry

        lax.fori_loop(0, JROWS, fire, 0)

    for n in range(NCTX):

        def drain(j, carry, n=n):
            pltpu.make_async_copy(
                m_hbm.at[idx_v.at[n * JROWS + j]],
                vals_v.at[pl.ds(n * PAIRS_W + j * ROW, ROW)],
                sem,
            ).wait()
            return carry

        lax.fori_loop(0, JROWS, drain, 0)

    for n in range(NCTX):
        pltpu.async_copy(
            vals_v.at[pl.ds(n * PAIRS_W, PAIRS_W)],
            out_hbm.at[pl.ds(n * B + base_b, PAIRS_W)],
            sem_out,
        )
    for n in range(NCTX):
        pltpu.make_async_copy(
            vals_v.at[pl.ds(n * PAIRS_W, PAIRS_W)],
            out_hbm.at[pl.ds(n * B + base_b, PAIRS_W)],
            sem_out,
        ).wait()


def kernel(pair_0, pair_1, target_table, context_table):
    m2 = _pairwise_dots(context_table.T, target_table.T)
    p0_flat = pair_0.reshape(-1)
    p1_t = pair_1.transpose(1, 2, 0).reshape(-1)
    out_t = _sc_gather(p0_flat, p1_t, m2.reshape(-1))
    return out_t.reshape(NCTX, B).T


# speedup vs baseline: 16.5531x; 1.0023x over previous
import functools

import jax
import jax.numpy as jnp
from jax import lax
from jax.experimental import pallas as pl
from jax.experimental.pallas import tpu as pltpu
from jax.experimental.pallas import tpu_sc as plsc

V = 1000
D = 64
B = 16384
NCTX = 5

GROW = 128
NG = 8

NUM_SC = 2
NUM_TEC = 16
NW = NUM_SC * NUM_TEC
PAIRS_W = B // NW
OUT_W = PAIRS_W * NCTX
ROW = 128
JROWS = PAIRS_W // ROW


def _matmul_body(ct_ref, tt_ref, m_ref):
    c = ct_ref[...]
    for g in range(NG):
        w = min(GROW, V - g * GROW)
        t_g = tt_ref[:, g * GROW : g * GROW + w]
        m_ref[pl.ds(g * V, V), 0:w] = lax.dot_general(
            c,
            t_g,
            dimension_numbers=(((0,), (0,)), ((), ())),
            preferred_element_type=jnp.float32,
        )


def _pairwise_dots(context_table_t, target_table_t):
    return pl.pallas_call(
        _matmul_body,
        out_shape=jax.ShapeDtypeStruct((NG * V, GROW), jnp.float32),
    )(context_table_t, target_table_t)


_sc_mesh = plsc.VectorSubcoreMesh(core_axis_name="c", subcore_axis_name="s")


@functools.partial(
    pl.kernel,
    out_type=jax.ShapeDtypeStruct((NCTX * B // ROW, ROW), jnp.float32),
    mesh=_sc_mesh,
    scratch_types=[
        pltpu.VMEM((PAIRS_W,), jnp.int32),
        pltpu.VMEM((OUT_W,), jnp.int32),
        pltpu.VMEM((NCTX * JROWS, ROW), jnp.int32),
        pltpu.VMEM((NCTX * JROWS, ROW), jnp.float32),
        pltpu.SemaphoreType.DMA,
        pltpu.SemaphoreType.DMA,
        pltpu.SemaphoreType.DMA,
    ],
)
def _sc_gather(p0_hbm, p1t_hbm, m_hbm, out_hbm, p0_v, p1_v, idx_v, vals_v,
               sem, sem_in, sem_out):
    wid = lax.axis_index("s") * NUM_SC + lax.axis_index("c")
    base_b = wid * PAIRS_W
    pltpu.async_copy(p0_hbm.at[pl.ds(base_b, PAIRS_W)], p0_v, sem_in)
    for n in range(NCTX):
        pltpu.async_copy(
            p1t_hbm.at[pl.ds(n * B + base_b, PAIRS_W)],
            p1_v.at[pl.ds(n * PAIRS_W, PAIRS_W)],
            sem_in,
        )
    pltpu.make_async_copy(p0_hbm.at[pl.ds(base_b, PAIRS_W)], p0_v, sem_in).wait()
    for n in range(NCTX):
        pltpu.make_async_copy(
            p1t_hbm.at[pl.ds(n * B + base_b, PAIRS_W)],
            p1_v.at[pl.ds(n * PAIRS_W, PAIRS_W)],
            sem_in,
        ).wait()

    for n in range(NCTX):

        def build(j, carry, n=n):
            for c in range(ROW // 16):
                k0 = j * ROW + c * 16
                it = p0_v[pl.ds(k0, 16)]
                jc = p1_v[pl.ds(n * PAIRS_W + k0, 16)]
                idx_v[n * JROWS + j, pl.ds(c * 16, 16)] = (
                    lax.shift_right_logical(it, 7) * (V * GROW)
                    + jc * GROW
                    + lax.bitwise_and(it, GROW - 1)
                )
            pltpu.async_copy(
                m_hbm.at[idx_v.at[n * JROWS + j]],
                vals_v.at[n * JROWS + j],
                sem,
            )
            return carry

        lax.fori_loop(0, JROWS, build, 0)

    for n in range(NCTX):

        def drain(j, carry, n=n):
            pltpu.make_async_copy(
                m_hbm.at[idx_v.at[n * JROWS + j]],
                vals_v.at[n * JROWS + j],
                sem,
            ).wait()
            return carry

        lax.fori_loop(0, JROWS, drain, 0)

    for n in range(NCTX):
        pltpu.async_copy(
            vals_v.at[pl.ds(n * JROWS, JROWS)],
            out_hbm.at[pl.ds(n * (B // ROW) + wid * JROWS, JROWS)],
            sem_out,
        )
    for n in range(NCTX):
        pltpu.make_async_copy(
            vals_v.at[pl.ds(n * JROWS, JROWS)],
            out_hbm.at[pl.ds(n * (B // ROW) + wid * JROWS, JROWS)],
            sem_out,
        ).wait()


def kernel(pair_0, pair_1, target_table, context_table):
    m2 = _pairwise_dots(context_table.T, target_table.T)
    p0_flat = pair_0.reshape(-1)
    p1_t = pair_1.transpose(1, 2, 0).reshape(-1)
    out_t = _sc_gather(p0_flat, p1_t, m2.reshape(-1))
    return out_t.reshape(NCTX, B).T
